# Initial kernel scaffold; baseline (speedup 1.0000x reference)
import jax
import jax.numpy as jnp
from jax.experimental import pallas as pl


def kernel(input1, input2):
    raise NotImplementedError("write your pallas kernel here")


# speedup vs baseline: 4.6261x; 4.6261x over previous
import functools

import jax
import jax.numpy as jnp
from jax.experimental import pallas as pl

_NEWTON_ITERS = 16


def _csparsemax_block(z_ref, u_ref, out_ref):
    z = z_ref[...]
    u = u_ref[...]
    lo = jnp.min(z - u, axis=-1, keepdims=True) - 1.0
    hi = jnp.max(z, axis=-1, keepdims=True)
    tau = 0.5 * (lo + hi)

    def body(_, carry):
        lo, hi, tau = carry
        t = z - tau
        free = (t > 0.0) & (t < u)
        clipped = t >= u
        nA = jnp.sum(free.astype(z.dtype), axis=-1, keepdims=True)
        s = jnp.sum(jnp.where(free, z, jnp.where(clipped, u, 0.0)),
                    axis=-1, keepdims=True)
        f = s - nA * tau - 1.0
        lo = jnp.where(f > 0.0, tau, lo)
        hi = jnp.where(f > 0.0, hi, tau)
        tau_n = (s - 1.0) / jnp.maximum(nA, 1.0)
        good = (nA > 0.0) & (((tau_n > lo) & (tau_n < hi)) | (tau_n == tau))
        tau = jnp.where(good, tau_n, 0.5 * (lo + hi))
        return (lo, hi, tau)

    lo, hi, tau = jax.lax.fori_loop(0, _NEWTON_ITERS, body, (lo, hi, tau),
                                    unroll=True)

    t = z - tau
    free = (t > 0.0) & (t < u)
    clipped = t >= u
    nA = jnp.sum(free.astype(z.dtype), axis=-1, keepdims=True)
    sumA = jnp.sum(jnp.where(free, z, 0.0), axis=-1, keepdims=True)
    sumB = jnp.sum(jnp.where(clipped, u, 0.0), axis=-1, keepdims=True)
    tau_f = (sumA + sumB - 1.0) / jnp.maximum(nA, 1.0)
    tau_f = jnp.where(nA > 0.0, tau_f, tau)
    out_ref[...] = jnp.where(free, z - tau_f,
                             jnp.where(clipped, u, jnp.zeros_like(z)))


@functools.partial(jax.jit, static_argnames=("block_rows",))
def _csparsemax(z, u, block_rows=16):
    rows, n = z.shape
    grid = (rows // block_rows,)
    spec = pl.BlockSpec((block_rows, n), lambda i: (i, 0))
    return pl.pallas_call(
        _csparsemax_block,
        grid=grid,
        in_specs=[spec, spec],
        out_specs=spec,
        out_shape=jax.ShapeDtypeStruct(z.shape, z.dtype),
    )(z, u)


def kernel(input1, input2):
    return _csparsemax(input1, input2)


# speedup vs baseline: 7.8837x; 1.7042x over previous
import functools

import jax
import jax.numpy as jnp
from jax.experimental import pallas as pl

_BISECT_ITERS = 10
_NEWTON_ITERS = 4


def _csparsemax_block(z_ref, u_ref, out_ref):
    z = z_ref[...]
    u = u_ref[...]
    lo = jnp.min(z - u, axis=-1, keepdims=True) - 1.0
    hi = jnp.max(z, axis=-1, keepdims=True)

    def bisect(_, carry):
        lo, hi = carry
        mid = 0.5 * (lo + hi)
        s = jnp.sum(jnp.clip(z - mid, 0.0, u), axis=-1, keepdims=True)
        pos = s > 1.0
        lo = jnp.where(pos, mid, lo)
        hi = jnp.where(pos, hi, mid)
        return (lo, hi)

    lo, hi = jax.lax.fori_loop(0, _BISECT_ITERS, bisect, (lo, hi), unroll=True)
    tau = 0.5 * (lo + hi)

    def newton(_, carry):
        lo, hi, tau = carry
        t = z - tau
        ltu = t < u
        free = (t > 0.0) & ltu
        nA = jnp.sum(free.astype(z.dtype), axis=-1, keepdims=True)
        s = jnp.sum(jnp.where(free, z, jnp.where(ltu, 0.0, u)),
                    axis=-1, keepdims=True)
        f = s - nA * tau - 1.0
        lo = jnp.where(f > 0.0, tau, lo)
        hi = jnp.where(f > 0.0, hi, tau)
        tau_n = (s - 1.0) / jnp.maximum(nA, 1.0)
        good = (nA > 0.0) & (((tau_n > lo) & (tau_n < hi)) | (tau_n == tau))
        tau = jnp.where(good, tau_n, 0.5 * (lo + hi))
        return (lo, hi, tau)

    lo, hi, tau = jax.lax.fori_loop(0, _NEWTON_ITERS, newton, (lo, hi, tau),
                                    unroll=True)

    t = z - tau
    ltu = t < u
    free = (t > 0.0) & ltu
    nA = jnp.sum(free.astype(z.dtype), axis=-1, keepdims=True)
    sAB = jnp.sum(jnp.where(free, z, jnp.where(ltu, 0.0, u)),
                  axis=-1, keepdims=True)
    tau_f = (sAB - 1.0) / jnp.maximum(nA, 1.0)
    tau_f = jnp.where(nA > 0.0, tau_f, tau)
    out_ref[...] = jnp.where(free, z - tau_f, jnp.where(ltu, 0.0, u))


@functools.partial(jax.jit, static_argnames=("block_rows",))
def _csparsemax(z, u, block_rows=16):
    rows, n = z.shape
    grid = (rows // block_rows,)
    spec = pl.BlockSpec((block_rows, n), lambda i: (i, 0))
    return pl.pallas_call(
        _csparsemax_block,
        grid=grid,
        in_specs=[spec, spec],
        out_specs=spec,
        out_shape=jax.ShapeDtypeStruct(z.shape, z.dtype),
    )(z, u)


def kernel(input1, input2):
    return _csparsemax(input1, input2)


# speedup vs baseline: 9.3853x; 1.1905x over previous
import functools

import jax
import jax.numpy as jnp
from jax.experimental import pallas as pl

_BISECT_ITERS = 13
_SECANT_ITERS = 2


def _csparsemax_block(z_ref, u_ref, out_ref):
    z = z_ref[...]
    u = u_ref[...]
    lo = jnp.min(z - u, axis=-1, keepdims=True) - 1.0
    hi = jnp.max(z, axis=-1, keepdims=True)

    def eval_f(tau):
        return jnp.sum(jnp.clip(z - tau, 0.0, u), axis=-1, keepdims=True) - 1.0

    tau_p = jnp.zeros_like(lo)
    f_p = jnp.zeros_like(lo)
    tau_c = jnp.zeros_like(lo)
    f_c = jnp.zeros_like(lo)
    for _ in range(_BISECT_ITERS):
        mid = 0.5 * (lo + hi)
        fm = eval_f(mid)
        pos = fm > 0.0
        lo = jnp.where(pos, mid, lo)
        hi = jnp.where(pos, hi, mid)
        tau_p, f_p = tau_c, f_c
        tau_c, f_c = mid, fm

    for _ in range(_SECANT_ITERS):
        denom = f_c - f_p
        ok = denom != 0.0
        tau_s = tau_c - f_c * (tau_c - tau_p) / jnp.where(ok, denom, 1.0)
        good = ok & (tau_s > lo) & (tau_s < hi)
        tau_n = jnp.where(good, tau_s, 0.5 * (lo + hi))
        fn = eval_f(tau_n)
        pos = fn > 0.0
        lo = jnp.where(pos, tau_n, lo)
        hi = jnp.where(pos, hi, tau_n)
        tau_p, f_p = tau_c, f_c
        tau_c, f_c = tau_n, fn

    t = z - tau_c
    free = (t > 0.0) & (t < u)
    nA = jnp.sum(free.astype(z.dtype), axis=-1, keepdims=True)
    C = jnp.sum(jnp.clip(t, 0.0, u), axis=-1, keepdims=True)
    f = C - 1.0
    pos = f > 0.0
    lo = jnp.where(pos, tau_c, lo)
    hi = jnp.where(pos, hi, tau_c)
    tau_s = tau_c + f / jnp.maximum(nA, 1.0)
    good = (nA > 0.0) & (((tau_s > lo) & (tau_s < hi)) | (tau_s == tau_c))
    tau = jnp.where(good, tau_s, 0.5 * (lo + hi))

    t = z - tau
    ltu = t < u
    free = (t > 0.0) & ltu
    nA = jnp.sum(free.astype(z.dtype), axis=-1, keepdims=True)
    sAB = jnp.sum(jnp.where(free, z, jnp.where(ltu, 0.0, u)),
                  axis=-1, keepdims=True)
    tau_f = (sAB - 1.0) / jnp.maximum(nA, 1.0)
    tau_f = jnp.where(nA > 0.0, tau_f, tau)
    out_ref[...] = jnp.where(free, z - tau_f, jnp.where(ltu, 0.0, u))


@functools.partial(jax.jit, static_argnames=("block_rows",))
def _csparsemax(z, u, block_rows=16):
    rows, n = z.shape
    grid = (rows // block_rows,)
    spec = pl.BlockSpec((block_rows, n), lambda i: (i, 0))
    return pl.pallas_call(
        _csparsemax_block,
        grid=grid,
        in_specs=[spec, spec],
        out_specs=spec,
        out_shape=jax.ShapeDtypeStruct(z.shape, z.dtype),
    )(z, u)


def kernel(input1, input2):
    return _csparsemax(input1, input2)


# speedup vs baseline: 9.8724x; 1.0519x over previous
import functools

import jax
import jax.numpy as jnp
from jax.experimental import pallas as pl

_BISECT_ITERS = 13
_SECANT_ITERS = 2


def _csparsemax_block(z_ref, u_ref, out_ref):
    z = z_ref[...]
    u = u_ref[...]
    lo = jnp.min(z - u, axis=-1, keepdims=True) - 1.0
    hi = jnp.max(z, axis=-1, keepdims=True)

    def eval_f(tau):
        return jnp.sum(jnp.clip(z - tau, 0.0, u), axis=-1, keepdims=True) - 1.0

    tau_p = jnp.zeros_like(lo)
    f_p = jnp.zeros_like(lo)
    tau_c = jnp.zeros_like(lo)
    f_c = jnp.zeros_like(lo)
    for _ in range(_BISECT_ITERS):
        mid = 0.5 * (lo + hi)
        fm = eval_f(mid)
        pos = fm > 0.0
        lo = jnp.where(pos, mid, lo)
        hi = jnp.where(pos, hi, mid)
        tau_p, f_p = tau_c, f_c
        tau_c, f_c = mid, fm

    for _ in range(_SECANT_ITERS):
        denom = f_c - f_p
        ok = denom != 0.0
        tau_s = tau_c - f_c * (tau_c - tau_p) / jnp.where(ok, denom, 1.0)
        good = ok & (tau_s > lo) & (tau_s < hi)
        tau_n = jnp.where(good, tau_s, 0.5 * (lo + hi))
        fn = eval_f(tau_n)
        pos = fn > 0.0
        lo = jnp.where(pos, tau_n, lo)
        hi = jnp.where(pos, hi, tau_n)
        tau_p, f_p = tau_c, f_c
        tau_c, f_c = tau_n, fn

    t = z - tau_c
    free = (t > 0.0) & (t < u)
    nA = jnp.sum(free.astype(z.dtype), axis=-1, keepdims=True)
    C = jnp.sum(jnp.clip(t, 0.0, u), axis=-1, keepdims=True)
    f = C - 1.0
    pos = f > 0.0
    lo = jnp.where(pos, tau_c, lo)
    hi = jnp.where(pos, hi, tau_c)
    tau_s = tau_c + f / jnp.maximum(nA, 1.0)
    good = (nA > 0.0) & (((tau_s > lo) & (tau_s < hi)) | (tau_s == tau_c))
    tau = jnp.where(good, tau_s, 0.5 * (lo + hi))

    t = z - tau
    ltu = t < u
    free = (t > 0.0) & ltu
    nA = jnp.sum(free.astype(z.dtype), axis=-1, keepdims=True)
    sAB = jnp.sum(jnp.where(free, z, jnp.where(ltu, 0.0, u)),
                  axis=-1, keepdims=True)
    tau_f = (sAB - 1.0) / jnp.maximum(nA, 1.0)
    tau_f = jnp.where(nA > 0.0, tau_f, tau)
    out_ref[...] = jnp.where(free, z - tau_f, jnp.where(ltu, 0.0, u))


@functools.partial(jax.jit, static_argnames=("block_rows",))
def _csparsemax(z, u, block_rows=32):
    rows, n = z.shape
    grid = (rows // block_rows,)
    spec = pl.BlockSpec((block_rows, n), lambda i: (i, 0))
    return pl.pallas_call(
        _csparsemax_block,
        grid=grid,
        in_specs=[spec, spec],
        out_specs=spec,
        out_shape=jax.ShapeDtypeStruct(z.shape, z.dtype),
    )(z, u)


def kernel(input1, input2):
    return _csparsemax(input1, input2)
